# Initial kernel scaffold; baseline (speedup 1.0000x reference)
import jax
import jax.numpy as jnp
from jax.experimental import pallas as pl


def kernel(x, emb_proton, emb_neutron, W1, b1, W2, b2):
    raise NotImplementedError("write your pallas kernel here")


# speedup vs baseline: 2.9061x; 2.9061x over previous
import functools

import jax
import jax.numpy as jnp
from jax import lax
from jax.experimental import pallas as pl
from jax.experimental.pallas import tpu as pltpu
from jax.experimental.pallas import tpu_sc as plsc

BATCH = 16384
DIM = 128
HIDDEN = 64
NUM_CORES = 2
NUM_SUBCORES = 16
NUM_WORKERS = NUM_CORES * NUM_SUBCORES
B_PER_W = BATCH // NUM_WORKERS


def _sc_gather(emb_p, emb_n, idx_p, idx_n):
  mesh = plsc.VectorSubcoreMesh(core_axis_name="c", subcore_axis_name="s")

  @functools.partial(
      pl.kernel,
      mesh=mesh,
      out_type=[
          jax.ShapeDtypeStruct((BATCH, DIM), jnp.float32),
          jax.ShapeDtypeStruct((BATCH, DIM), jnp.float32),
      ],
      scratch_types=[
          pltpu.VMEM((B_PER_W,), jnp.int32),
          pltpu.VMEM((B_PER_W, DIM), jnp.float32),
          pltpu.SemaphoreType.DMA,
      ],
  )
  def gather_kernel(embp_hbm, embn_hbm, idxp_hbm, idxn_hbm,
                    outp_hbm, outn_hbm, idx_v, rows_v, sem):
    wid = lax.axis_index("s") * NUM_CORES + lax.axis_index("c")
    base = wid * B_PER_W
    pltpu.sync_copy(idxp_hbm.at[pl.ds(base, B_PER_W)], idx_v)
    pltpu.async_copy(embp_hbm.at[idx_v], rows_v, sem).wait()
    pltpu.sync_copy(rows_v, outp_hbm.at[pl.ds(base, B_PER_W)])
    pltpu.sync_copy(idxn_hbm.at[pl.ds(base, B_PER_W)], idx_v)
    pltpu.async_copy(embn_hbm.at[idx_v], rows_v, sem).wait()
    pltpu.sync_copy(rows_v, outn_hbm.at[pl.ds(base, B_PER_W)])

  return gather_kernel(emb_p, emb_n, idx_p, idx_n)


def _tc_mlp(p_rows, n_rows, w1p_t, w1n_t, b1_row, w2_t, b2_11):
  bm = 2048
  grid = (BATCH // bm,)

  def body(p_ref, n_ref, w1p_ref, w1n_ref, b1_ref, w2_ref, b2_ref, o_ref):
    h = jnp.dot(p_ref[...], w1p_ref[...], preferred_element_type=jnp.float32)
    h = h + jnp.dot(n_ref[...], w1n_ref[...],
                    preferred_element_type=jnp.float32)
    h = jnp.maximum(h + b1_ref[...], 0.0)
    o_ref[...] = jnp.dot(h, w2_ref[...],
                         preferred_element_type=jnp.float32) + b2_ref[...]

  return pl.pallas_call(
      body,
      grid=grid,
      in_specs=[
          pl.BlockSpec((bm, DIM), lambda i: (i, 0)),
          pl.BlockSpec((bm, DIM), lambda i: (i, 0)),
          pl.BlockSpec((DIM, HIDDEN), lambda i: (0, 0)),
          pl.BlockSpec((DIM, HIDDEN), lambda i: (0, 0)),
          pl.BlockSpec((1, HIDDEN), lambda i: (0, 0)),
          pl.BlockSpec((HIDDEN, 1), lambda i: (0, 0)),
          pl.BlockSpec((1, 1), lambda i: (0, 0)),
      ],
      out_specs=pl.BlockSpec((bm, 1), lambda i: (i, 0)),
      out_shape=jax.ShapeDtypeStruct((BATCH, 1), jnp.float32),
  )(p_rows, n_rows, w1p_t, w1n_t, b1_row, w2_t, b2_11)


@jax.jit
def kernel(x, emb_proton, emb_neutron, W1, b1, W2, b2):
  idx = x.astype(jnp.int32)
  idx_p = idx[:, 0]
  idx_n = idx[:, 1]
  p_rows, n_rows = _sc_gather(emb_proton, emb_neutron, idx_p, idx_n)
  w1_t = W1.T
  w1p_t = w1_t[:DIM]
  w1n_t = w1_t[DIM:]
  b1_row = b1.reshape(1, HIDDEN)
  w2_t = W2.T
  b2_11 = b2.reshape(1, 1)
  return _tc_mlp(p_rows, n_rows, w1p_t, w1n_t, b1_row, w2_t, b2_11)
